# Initial kernel scaffold; baseline (speedup 1.0000x reference)
import jax
import jax.numpy as jnp
from jax.experimental import pallas as pl


def kernel(x, edge_index, W_rel1, b_rel1, W_root1, gamma1, beta1, W_rel2, b_rel2, W_root2, gamma2, beta2, W_rel3, b_rel3, W_root3):
    raise NotImplementedError("write your pallas kernel here")


# speedup vs baseline: 2.9824x; 2.9824x over previous
import functools

import jax
import jax.numpy as jnp
from jax import lax
from jax.experimental import pallas as pl
from jax.experimental.pallas import tpu as pltpu
from jax.experimental.pallas import tpu_sc as plsc

N_NODES = 10000
N_EDGES = 320000
D_IN = 128
D_HID = 128
D_OUT = 64

NC = 2
NS = 16
NW = NC * NS
CHUNK = 128
GROUP = 8
EPT = -(-N_EDGES // NW // (CHUNK * GROUP)) * CHUNK * GROUP
NCHUNK = EPT // CHUNK
NGROUP = NCHUNK // GROUP
E_PAD = NW * EPT
TRASH = N_NODES
NT = 10112
ROWS_PER_TILE = NT // NS

ROW_BLK = 1000
GRID = N_NODES // ROW_BLK


def _make_sc_aggregate(d):
    mesh = plsc.VectorSubcoreMesh(core_axis_name="c", subcore_axis_name="s")

    @functools.partial(
        pl.kernel,
        mesh=mesh,
        out_type=jax.ShapeDtypeStruct((NC, NT, d), jnp.float32),
        scratch_types=[
            pltpu.VMEM_SHARED((NT, d), jnp.float32),
            pltpu.VMEM((GROUP, CHUNK), jnp.int32),
            pltpu.VMEM((GROUP, CHUNK), jnp.int32),
            pltpu.VMEM((CHUNK, d), jnp.float32),
            pltpu.VMEM((CHUNK, d), jnp.float32),
            pltpu.SemaphoreType.DMA,
            pltpu.SemaphoreType.DMA,
        ],
    )
    def agg(y_hbm, src_hbm, dst_hbm, zeros_hbm, out_hbm,
            table, src_v, dst_v, buf0, buf1, sem0, sem1):
        cid = lax.axis_index("c")
        sid = lax.axis_index("s")
        wid = cid * NS + sid

        z0 = sid * ROWS_PER_TILE
        pltpu.sync_copy(zeros_hbm.at[pl.ds(z0, ROWS_PER_TILE)],
                        table.at[pl.ds(z0, ROWS_PER_TILE)])
        plsc.subcore_barrier()

        def group_body(g, _):
            g0 = g * GROUP
            pltpu.sync_copy(src_hbm.at[wid, pl.ds(g0, GROUP)], src_v)
            pltpu.sync_copy(dst_hbm.at[wid, pl.ds(g0, GROUP)], dst_v)
            pltpu.async_copy(y_hbm.at[src_v.at[0]], buf0, sem0)
            for j in range(GROUP):
                buf, sem = (buf0, sem0) if j % 2 == 0 else (buf1, sem1)
                if j + 1 < GROUP:
                    nbuf, nsem = (buf1, sem1) if j % 2 == 0 else (buf0, sem0)
                    pltpu.async_copy(y_hbm.at[src_v.at[j + 1]], nbuf, nsem)
                pltpu.make_async_copy(y_hbm.at[src_v.at[j]], buf, sem).wait()
                pltpu.sync_copy(buf, table.at[dst_v.at[j]], add=True)
            return 0

        lax.fori_loop(0, NGROUP, group_body, 0)
        plsc.subcore_barrier()

        pltpu.sync_copy(table.at[pl.ds(z0, ROWS_PER_TILE)],
                        out_hbm.at[cid, pl.ds(z0, ROWS_PER_TILE)])

    return agg


_agg128 = _make_sc_aggregate(D_HID)


def _dot(a, b):
    return lax.dot_general(a, b, (((1,), (0,)), ((), ())),
                           preferred_element_type=jnp.float32)


def _mm_body(h_ref, w_ref, y_ref, r_ref, *, d):
    hw = _dot(h_ref[...], w_ref[...])
    y_ref[...] = hw[:, :d]
    r_ref[...] = hw[:, d:]


def _mm(h, w_cat, d):
    din = h.shape[1]
    dtot = w_cat.shape[1]
    return pl.pallas_call(
        functools.partial(_mm_body, d=d),
        grid=(GRID,),
        in_specs=[
            pl.BlockSpec((ROW_BLK, din), lambda i: (i, 0)),
            pl.BlockSpec((din, dtot), lambda i: (0, 0)),
        ],
        out_specs=[
            pl.BlockSpec((ROW_BLK, d), lambda i: (i, 0)),
            pl.BlockSpec((ROW_BLK, dtot - d), lambda i: (i, 0)),
        ],
        out_shape=[
            jax.ShapeDtypeStruct((N_NODES, d), jnp.float32),
            jax.ShapeDtypeStruct((N_NODES, dtot - d), jnp.float32),
        ],
    )(h, w_cat)


def _comb_body(p_ref, r_ref, b_ref, z_ref, s_ref):
    i = pl.program_id(0)
    z = p_ref[0] + p_ref[1] + r_ref[...] + b_ref[...]
    z_ref[...] = z

    @pl.when(i == 0)
    def _():
        s_ref[...] = jnp.zeros_like(s_ref)

    s_ref[0:1, :] += jnp.sum(z, axis=0, keepdims=True)
    s_ref[1:2, :] += jnp.sum(z * z, axis=0, keepdims=True)


def _comb(p, r, b):
    d = r.shape[1]
    return pl.pallas_call(
        _comb_body,
        grid=(GRID,),
        in_specs=[
            pl.BlockSpec((NC, ROW_BLK, d), lambda i: (0, i, 0)),
            pl.BlockSpec((ROW_BLK, d), lambda i: (i, 0)),
            pl.BlockSpec((1, d), lambda i: (0, 0)),
        ],
        out_specs=[
            pl.BlockSpec((ROW_BLK, d), lambda i: (i, 0)),
            pl.BlockSpec((2, d), lambda i: (0, 0)),
        ],
        out_shape=[
            jax.ShapeDtypeStruct((N_NODES, d), jnp.float32),
            jax.ShapeDtypeStruct((2, d), jnp.float32),
        ],
    )(p, r, b.reshape(1, d))


def _bn_mm_body(z_ref, s_ref, g_ref, be_ref, w_ref, y_ref, r_ref, *, d):
    inv_n = 1.0 / N_NODES
    mu = s_ref[0:1, :] * inv_n
    var = s_ref[1:2, :] * inv_n - mu * mu
    scale = g_ref[...] * lax.rsqrt(var + 1e-5)
    h = jnp.maximum((z_ref[...] - mu) * scale + be_ref[...], 0.0)
    hw = _dot(h, w_ref[...])
    y_ref[...] = hw[:, :d]
    r_ref[...] = hw[:, d:]


def _bn_mm(z, s, gamma, beta, w_cat, d):
    din = z.shape[1]
    dtot = w_cat.shape[1]
    return pl.pallas_call(
        functools.partial(_bn_mm_body, d=d),
        grid=(GRID,),
        in_specs=[
            pl.BlockSpec((ROW_BLK, din), lambda i: (i, 0)),
            pl.BlockSpec((2, din), lambda i: (0, 0)),
            pl.BlockSpec((1, din), lambda i: (0, 0)),
            pl.BlockSpec((1, din), lambda i: (0, 0)),
            pl.BlockSpec((din, dtot), lambda i: (0, 0)),
        ],
        out_specs=[
            pl.BlockSpec((ROW_BLK, d), lambda i: (i, 0)),
            pl.BlockSpec((ROW_BLK, dtot - d), lambda i: (i, 0)),
        ],
        out_shape=[
            jax.ShapeDtypeStruct((N_NODES, d), jnp.float32),
            jax.ShapeDtypeStruct((N_NODES, dtot - d), jnp.float32),
        ],
    )(z, s, gamma.reshape(1, din), beta.reshape(1, din), w_cat)


def _final_body(p_ref, r_ref, b_ref, o_ref, *, d):
    z = p_ref[0][:, :d] + p_ref[1][:, :d] + r_ref[...] + b_ref[...]
    m = jnp.max(z, axis=-1, keepdims=True)
    lse = jnp.log(jnp.sum(jnp.exp(z - m), axis=-1, keepdims=True)) + m
    o_ref[...] = z - lse


def _final(p, r, b):
    d = r.shape[1]
    dp = p.shape[2]
    return pl.pallas_call(
        functools.partial(_final_body, d=d),
        grid=(GRID,),
        in_specs=[
            pl.BlockSpec((NC, ROW_BLK, dp), lambda i: (0, i, 0)),
            pl.BlockSpec((ROW_BLK, d), lambda i: (i, 0)),
            pl.BlockSpec((1, d), lambda i: (0, 0)),
        ],
        out_specs=pl.BlockSpec((ROW_BLK, d), lambda i: (i, 0)),
        out_shape=jax.ShapeDtypeStruct((N_NODES, d), jnp.float32),
    )(p, r, b.reshape(1, d))


def kernel(x, edge_index, W_rel1, b_rel1, W_root1, gamma1, beta1,
           W_rel2, b_rel2, W_root2, gamma2, beta2,
           W_rel3, b_rel3, W_root3):
    src = edge_index[0].astype(jnp.int32)
    dst = edge_index[1].astype(jnp.int32)
    pad = E_PAD - N_EDGES
    srcs = jnp.concatenate([src, jnp.zeros((pad,), jnp.int32)])
    dsts = jnp.concatenate([dst, jnp.full((pad,), TRASH, jnp.int32)])
    srcs = srcs.reshape(NW, NCHUNK, CHUNK)
    dsts = dsts.reshape(NW, NCHUNK, CHUNK)
    zeros128 = jnp.zeros((NT, D_HID), jnp.float32)

    wc1 = jnp.concatenate([W_rel1, W_root1], axis=1)
    wc2 = jnp.concatenate([W_rel2, W_root2], axis=1)
    wc3 = jnp.concatenate(
        [W_rel3, jnp.zeros((D_HID, D_HID - D_OUT), jnp.float32), W_root3],
        axis=1)

    y1, r1 = _mm(x, wc1, D_HID)
    p1 = _agg128(y1, srcs, dsts, zeros128)
    z1, s1 = _comb(p1, r1, b_rel1)

    y2, r2 = _bn_mm(z1, s1, gamma1, beta1, wc2, D_HID)
    p2 = _agg128(y2, srcs, dsts, zeros128)
    z2, s2 = _comb(p2, r2, b_rel2)

    y3, r3 = _bn_mm(z2, s2, gamma2, beta2, wc3, D_HID)
    p3 = _agg128(y3, srcs, dsts, zeros128)
    return _final(p3, r3, b_rel3)


# speedup vs baseline: 3.6446x; 1.2220x over previous
import functools

import jax
import jax.numpy as jnp
from jax import lax
from jax.experimental import pallas as pl
from jax.experimental.pallas import tpu as pltpu
from jax.experimental.pallas import tpu_sc as plsc

N_NODES = 10000
N_EDGES = 320000
D_IN = 128
D_HID = 128
D_OUT = 64

NC = 2
NS = 16
NW = NC * NS
CHUNK = 128
GROUP = 8
NG_TOTAL = 20
NG_A = 16
NG_B = NG_TOTAL - NG_A
E_PAD = NS * NG_TOTAL * GROUP * CHUNK
NROWS = E_PAD // CHUNK
TRASH = N_NODES
NT = 10112
ROWS_PER_TILE = NT // NS

ROW_BLK = 1000
GRID = N_NODES // ROW_BLK


def _make_sc_aggregate(d):
    mesh = plsc.VectorSubcoreMesh(core_axis_name="c", subcore_axis_name="s")

    @functools.partial(
        pl.kernel,
        mesh=mesh,
        out_type=jax.ShapeDtypeStruct((NC, NT, d), jnp.float32),
        scratch_types=[
            pltpu.VMEM_SHARED((NT, d), jnp.float32),
            pltpu.VMEM((GROUP, CHUNK), jnp.int32),
            pltpu.VMEM((GROUP, CHUNK), jnp.int32),
            pltpu.VMEM((CHUNK, d), jnp.float32),
            pltpu.VMEM((CHUNK, d), jnp.float32),
            pltpu.SemaphoreType.DMA,
            pltpu.SemaphoreType.DMA,
        ],
    )
    def agg(y_hbm, src_hbm, dst_hbm, zeros_hbm, out_hbm,
            table, src_v, dst_v, buf0, buf1, sem0, sem1):
        cid = lax.axis_index("c")
        sid = lax.axis_index("s")

        z0 = sid * ROWS_PER_TILE
        pltpu.sync_copy(zeros_hbm.at[pl.ds(z0, ROWS_PER_TILE)],
                        table.at[pl.ds(z0, ROWS_PER_TILE)])
        plsc.subcore_barrier()

        row0 = jnp.where(cid == 0, sid * NG_A,
                         NS * NG_A + sid * NG_B) * GROUP
        ng = jnp.where(cid == 0, NG_A, NG_B)

        def group_body(g, _):
            g0 = row0 + g * GROUP
            pltpu.sync_copy(src_hbm.at[pl.ds(g0, GROUP)], src_v)
            pltpu.sync_copy(dst_hbm.at[pl.ds(g0, GROUP)], dst_v)
            pltpu.async_copy(y_hbm.at[src_v.at[0]], buf0, sem0)
            for j in range(GROUP):
                buf, sem = (buf0, sem0) if j % 2 == 0 else (buf1, sem1)
                if j + 1 < GROUP:
                    nbuf, nsem = (buf1, sem1) if j % 2 == 0 else (buf0, sem0)
                    pltpu.async_copy(y_hbm.at[src_v.at[j + 1]], nbuf, nsem)
                pltpu.make_async_copy(y_hbm.at[src_v.at[j]], buf, sem).wait()
                pltpu.sync_copy(buf, table.at[dst_v.at[j]], add=True)
            return 0

        lax.fori_loop(0, ng, group_body, 0)
        plsc.subcore_barrier()

        pltpu.sync_copy(table.at[pl.ds(z0, ROWS_PER_TILE)],
                        out_hbm.at[cid, pl.ds(z0, ROWS_PER_TILE)])

    return agg


_agg128 = _make_sc_aggregate(D_HID)


def _dot(a, b):
    return lax.dot_general(a, b, (((1,), (0,)), ((), ())),
                           preferred_element_type=jnp.float32)


def _mm_body(h_ref, w_ref, y_ref, r_ref, *, d):
    hw = _dot(h_ref[...], w_ref[...])
    y_ref[...] = hw[:, :d]
    r_ref[...] = hw[:, d:]


def _mm(h, w_cat, d):
    din = h.shape[1]
    dtot = w_cat.shape[1]
    return pl.pallas_call(
        functools.partial(_mm_body, d=d),
        grid=(GRID,),
        in_specs=[
            pl.BlockSpec((ROW_BLK, din), lambda i: (i, 0)),
            pl.BlockSpec((din, dtot), lambda i: (0, 0)),
        ],
        out_specs=[
            pl.BlockSpec((ROW_BLK, d), lambda i: (i, 0)),
            pl.BlockSpec((ROW_BLK, dtot - d), lambda i: (i, 0)),
        ],
        out_shape=[
            jax.ShapeDtypeStruct((N_NODES, d), jnp.float32),
            jax.ShapeDtypeStruct((N_NODES, dtot - d), jnp.float32),
        ],
    )(h, w_cat)


def _comb_body(p_ref, r_ref, b_ref, z_ref, s_ref):
    i = pl.program_id(0)
    z = p_ref[0] + p_ref[1] + r_ref[...] + b_ref[...]
    z_ref[...] = z

    @pl.when(i == 0)
    def _():
        s_ref[...] = jnp.zeros_like(s_ref)

    s_ref[0:1, :] += jnp.sum(z, axis=0, keepdims=True)
    s_ref[1:2, :] += jnp.sum(z * z, axis=0, keepdims=True)


def _comb(p, r, b):
    d = r.shape[1]
    return pl.pallas_call(
        _comb_body,
        grid=(GRID,),
        in_specs=[
            pl.BlockSpec((NC, ROW_BLK, d), lambda i: (0, i, 0)),
            pl.BlockSpec((ROW_BLK, d), lambda i: (i, 0)),
            pl.BlockSpec((1, d), lambda i: (0, 0)),
        ],
        out_specs=[
            pl.BlockSpec((ROW_BLK, d), lambda i: (i, 0)),
            pl.BlockSpec((2, d), lambda i: (0, 0)),
        ],
        out_shape=[
            jax.ShapeDtypeStruct((N_NODES, d), jnp.float32),
            jax.ShapeDtypeStruct((2, d), jnp.float32),
        ],
    )(p, r, b.reshape(1, d))


def _bn_mm_body(z_ref, s_ref, g_ref, be_ref, w_ref, y_ref, r_ref, *, d):
    inv_n = 1.0 / N_NODES
    mu = s_ref[0:1, :] * inv_n
    var = s_ref[1:2, :] * inv_n - mu * mu
    scale = g_ref[...] * lax.rsqrt(var + 1e-5)
    h = jnp.maximum((z_ref[...] - mu) * scale + be_ref[...], 0.0)
    hw = _dot(h, w_ref[...])
    y_ref[...] = hw[:, :d]
    r_ref[...] = hw[:, d:]


def _bn_mm(z, s, gamma, beta, w_cat, d):
    din = z.shape[1]
    dtot = w_cat.shape[1]
    return pl.pallas_call(
        functools.partial(_bn_mm_body, d=d),
        grid=(GRID,),
        in_specs=[
            pl.BlockSpec((ROW_BLK, din), lambda i: (i, 0)),
            pl.BlockSpec((2, din), lambda i: (0, 0)),
            pl.BlockSpec((1, din), lambda i: (0, 0)),
            pl.BlockSpec((1, din), lambda i: (0, 0)),
            pl.BlockSpec((din, dtot), lambda i: (0, 0)),
        ],
        out_specs=[
            pl.BlockSpec((ROW_BLK, d), lambda i: (i, 0)),
            pl.BlockSpec((ROW_BLK, dtot - d), lambda i: (i, 0)),
        ],
        out_shape=[
            jax.ShapeDtypeStruct((N_NODES, d), jnp.float32),
            jax.ShapeDtypeStruct((N_NODES, dtot - d), jnp.float32),
        ],
    )(z, s, gamma.reshape(1, din), beta.reshape(1, din), w_cat)


def _final_body(p_ref, r_ref, b_ref, o_ref, *, d):
    z = p_ref[0][:, :d] + p_ref[1][:, :d] + r_ref[...] + b_ref[...]
    m = jnp.max(z, axis=-1, keepdims=True)
    lse = jnp.log(jnp.sum(jnp.exp(z - m), axis=-1, keepdims=True)) + m
    o_ref[...] = z - lse


def _final(p, r, b):
    d = r.shape[1]
    dp = p.shape[2]
    return pl.pallas_call(
        functools.partial(_final_body, d=d),
        grid=(GRID,),
        in_specs=[
            pl.BlockSpec((NC, ROW_BLK, dp), lambda i: (0, i, 0)),
            pl.BlockSpec((ROW_BLK, d), lambda i: (i, 0)),
            pl.BlockSpec((1, d), lambda i: (0, 0)),
        ],
        out_specs=pl.BlockSpec((ROW_BLK, d), lambda i: (i, 0)),
        out_shape=jax.ShapeDtypeStruct((N_NODES, d), jnp.float32),
    )(p, r, b.reshape(1, d))


def kernel(x, edge_index, W_rel1, b_rel1, W_root1, gamma1, beta1,
           W_rel2, b_rel2, W_root2, gamma2, beta2,
           W_rel3, b_rel3, W_root3):
    src = edge_index[0].astype(jnp.int32)
    dst = edge_index[1].astype(jnp.int32)
    pad = E_PAD - N_EDGES
    srcs = jnp.concatenate([src, jnp.zeros((pad,), jnp.int32)])
    dsts = jnp.concatenate([dst, jnp.full((pad,), TRASH, jnp.int32)])
    srcs = srcs.reshape(NROWS, CHUNK)
    dsts = dsts.reshape(NROWS, CHUNK)
    zeros128 = jnp.zeros((NT, D_HID), jnp.float32)

    wc1 = jnp.concatenate([W_rel1, W_root1], axis=1)
    wc2 = jnp.concatenate([W_rel2, W_root2], axis=1)
    wc3 = jnp.concatenate(
        [W_rel3, jnp.zeros((D_HID, D_HID - D_OUT), jnp.float32), W_root3],
        axis=1)

    y1, r1 = _mm(x, wc1, D_HID)
    p1 = _agg128(y1, srcs, dsts, zeros128)
    z1, s1 = _comb(p1, r1, b_rel1)

    y2, r2 = _bn_mm(z1, s1, gamma1, beta1, wc2, D_HID)
    p2 = _agg128(y2, srcs, dsts, zeros128)
    z2, s2 = _comb(p2, r2, b_rel2)

    y3, r3 = _bn_mm(z2, s2, gamma2, beta2, wc3, D_HID)
    p3 = _agg128(y3, srcs, dsts, zeros128)
    return _final(p3, r3, b_rel3)
